# Initial kernel scaffold; baseline (speedup 1.0000x reference)
import jax
import jax.numpy as jnp
from jax.experimental import pallas as pl


def kernel(softmaxes, labels):
    raise NotImplementedError("write your pallas kernel here")


# speedup vs baseline: 1.2849x; 1.2849x over previous
import jax
import jax.numpy as jnp
from jax.experimental import pallas as pl

N_BINS = 50


def _ece_kernel(smax_ref, labels_ref, lowers_ref, uppers_ref,
                ece_ref, acc_ref, conf_ref, prob_ref):
    i = pl.program_id(0)
    n = pl.num_programs(0)

    block = smax_ref[...]
    C = block.shape[0]
    conf = jnp.max(block, axis=0)
    row_ids = jax.lax.broadcasted_iota(jnp.int32, block.shape, 0)
    pred = jnp.min(jnp.where(block == conf[None, :], row_ids, C), axis=0)
    labels = labels_ref[0, :]
    acc = (pred == labels).astype(jnp.float32)

    lowers = lowers_ref[...]
    uppers = uppers_ref[...]
    cb = conf[None, :]
    mask = ((cb > lowers) & (cb <= uppers)).astype(jnp.float32)
    prob_part = jnp.sum(mask, axis=1)
    acc_part = jnp.sum(mask * acc[None, :], axis=1)
    conf_part = jnp.sum(mask * cb, axis=1)

    @pl.when(i == 0)
    def _init():
        acc_ref[...] = jnp.zeros_like(acc_ref)
        conf_ref[...] = jnp.zeros_like(conf_ref)
        prob_ref[...] = jnp.zeros_like(prob_ref)
        ece_ref[...] = jnp.zeros_like(ece_ref)

    acc_ref[...] += acc_part[None, :]
    conf_ref[...] += conf_part[None, :]
    prob_ref[...] += prob_part[None, :]

    @pl.when(i == n - 1)
    def _finish():
        prob_bins = prob_ref[0, :]
        acc_bins = acc_ref[0, :]
        conf_bins = conf_ref[0, :]
        valid = prob_bins > 0
        safe = jnp.where(valid, prob_bins, 1.0)
        acc_n = jnp.where(valid, acc_bins / safe, 0.0)
        conf_n = jnp.where(valid, conf_bins / safe, 0.0)
        prob_n = prob_bins / jnp.sum(prob_bins)
        ece = jnp.sum(jnp.where(valid, jnp.abs(conf_n - acc_n) * prob_n, 0.0))
        ece_ref[...] = jnp.reshape(ece, (1, 1))


def kernel(softmaxes, labels):
    C, B = softmaxes.shape
    T = 2048
    grid = B // T

    bnd = jnp.linspace(0.0, 1.0, N_BINS + 1)
    lowers = bnd[:-1].reshape(N_BINS, 1)
    uppers = bnd[1:].reshape(N_BINS, 1)
    labels2 = labels.reshape(1, B)

    ece, acc_bins, conf_bins, prob_bins = pl.pallas_call(
        _ece_kernel,
        grid=(grid,),
        in_specs=[
            pl.BlockSpec((C, T), lambda i: (0, i)),
            pl.BlockSpec((1, T), lambda i: (0, i)),
            pl.BlockSpec((N_BINS, 1), lambda i: (0, 0)),
            pl.BlockSpec((N_BINS, 1), lambda i: (0, 0)),
        ],
        out_specs=[
            pl.BlockSpec((1, 1), lambda i: (0, 0)),
            pl.BlockSpec((1, N_BINS), lambda i: (0, 0)),
            pl.BlockSpec((1, N_BINS), lambda i: (0, 0)),
            pl.BlockSpec((1, N_BINS), lambda i: (0, 0)),
        ],
        out_shape=[
            jax.ShapeDtypeStruct((1, 1), jnp.float32),
            jax.ShapeDtypeStruct((1, N_BINS), jnp.float32),
            jax.ShapeDtypeStruct((1, N_BINS), jnp.float32),
            jax.ShapeDtypeStruct((1, N_BINS), jnp.float32),
        ],
    )(softmaxes, labels2, lowers, uppers)
    return (ece[0, 0], acc_bins[0], conf_bins[0], prob_bins[0])
